# baseline (speedup 1.0000x reference)
import dataclasses
import functools

import jax
import jax.numpy as jnp
from jax import lax
from jax.experimental import pallas as pl
from jax.experimental.pallas import tpu as pltpu
from jax.experimental.pallas import tpu_sc as plsc

_NC = 2
_NS = 16
_NW = _NC * _NS
_L = 16

_BATCH = 16384
_D = 16
_B_PER_W = _BATCH // _NW
_CHUNK = 128
_ROWS_PER_BIG = 128 // _D


def _compiler_params():
    cp = pltpu.CompilerParams()
    if "needs_layout_passes" in pltpu.CompilerParams.__dataclass_fields__:
        cp = dataclasses.replace(cp, needs_layout_passes=False)
    return cp


_C_BLK = 4096
_R_BLK = _C_BLK * _D // 128


def _tc_relayout(table_t):
    n_v = table_t.shape[1]
    grid = (n_v + _C_BLK - 1) // _C_BLK

    def body(i_ref, o_ref):
        o_ref[...] = pltpu.einshape(
            "d(rs)->r(sd)", i_ref[...], r=_R_BLK, s=_ROWS_PER_BIG)

    return pl.pallas_call(
        body,
        grid=(grid,),
        in_specs=[pl.BlockSpec((_D, _C_BLK), lambda i: (0, i))],
        out_specs=pl.BlockSpec((_R_BLK, 128), lambda i: (i, 0)),
        out_shape=jax.ShapeDtypeStruct((grid * _R_BLK, 128), jnp.float32),
        compiler_params=pltpu.CompilerParams(
            dimension_semantics=("arbitrary",)),
    )(table_t)


def kernel(user_ids, item_ids, user_table, item_table):
    ut_big = _tc_relayout(user_table.T)
    it_big = _tc_relayout(item_table.T)

    mesh = plsc.VectorSubcoreMesh(core_axis_name="c", subcore_axis_name="s")

    @functools.partial(
        pl.kernel,
        mesh=mesh,
        out_type=jax.ShapeDtypeStruct((_BATCH,), jnp.float32),
        scratch_types=[
            pltpu.VMEM((_B_PER_W,), jnp.int32),
            pltpu.VMEM((_B_PER_W,), jnp.int32),
            pltpu.VMEM((_B_PER_W,), jnp.int32),
            pltpu.VMEM((_B_PER_W,), jnp.int32),
            pltpu.VMEM((_CHUNK, 128), jnp.float32),
            pltpu.VMEM((_CHUNK, 128), jnp.float32),
            pltpu.VMEM((_B_PER_W,), jnp.float32),
            pltpu.SemaphoreType.DMA,
            pltpu.SemaphoreType.DMA,
        ],
        compiler_params=_compiler_params(),
    )
    def sc_kernel(uid_hbm, iid_hbm, ut_hbm, it_hbm, out_hbm,
                  uidx_v, iidx_v, ubig_v, ibig_v, u_rows, i_rows, out_v,
                  sem_u, sem_i):
        wid = lax.axis_index("s") * _NC + lax.axis_index("c")
        base = wid * _B_PER_W
        pltpu.sync_copy(uid_hbm.at[pl.ds(base, _B_PER_W)], uidx_v)
        pltpu.sync_copy(iid_hbm.at[pl.ds(base, _B_PER_W)], iidx_v)

        @pl.loop(0, _B_PER_W, step=_L)
        def _(k):
            ubig_v[pl.ds(k, _L)] = uidx_v[pl.ds(k, _L)] >> 3
            ibig_v[pl.ds(k, _L)] = iidx_v[pl.ds(k, _L)] >> 3

        lane = lax.iota(jnp.int32, _L)

        for c in range(_B_PER_W // _CHUNK):
            cu = pltpu.async_copy(
                ut_hbm.at[ubig_v.at[pl.ds(c * _CHUNK, _CHUNK)]], u_rows, sem_u)
            ci = pltpu.async_copy(
                it_hbm.at[ibig_v.at[pl.ds(c * _CHUNK, _CHUNK)]], i_rows, sem_i)
            cu.wait()
            ci.wait()

            @pl.loop(0, _CHUNK, step=_L)
            def _(g):
                j = g + lane
                uid = uidx_v[pl.ds(c * _CHUNK + g, _L)]
                iid = iidx_v[pl.ds(c * _CHUNK + g, _L)]
                ucol = (uid & (_ROWS_PER_BIG - 1)) * _D
                icol = (iid & (_ROWS_PER_BIG - 1)) * _D
                acc = jnp.zeros((_L,), jnp.float32)
                for dd in range(_D):
                    ug = plsc.load_gather(u_rows, [j, ucol + dd])
                    vg = plsc.load_gather(i_rows, [j, icol + dd])
                    acc = acc + ug * vg
                out_v[pl.ds(c * _CHUNK + g, _L)] = acc

        pltpu.sync_copy(out_v, out_hbm.at[pl.ds(base, _B_PER_W)])

    return sc_kernel(user_ids, item_ids, ut_big, it_big)


# speedup vs baseline: 2.1319x; 2.1319x over previous
import dataclasses
import functools

import jax
import jax.numpy as jnp
from jax import lax
from jax.experimental import pallas as pl
from jax.experimental.pallas import tpu as pltpu
from jax.experimental.pallas import tpu_sc as plsc

_NC = 2
_NS = 16
_NW = _NC * _NS
_L = 16

_BATCH = 16384
_D = 16
_B_PER_W = _BATCH // _NW


def _compiler_params():
    cp = pltpu.CompilerParams()
    fields = pltpu.CompilerParams.__dataclass_fields__
    if "needs_layout_passes" in fields:
        cp = dataclasses.replace(cp, needs_layout_passes=False)
    if "use_tc_tiling_on_sc" in fields:
        cp = dataclasses.replace(cp, use_tc_tiling_on_sc=False)
    return cp


def kernel(user_ids, item_ids, user_table, item_table):
    mesh = plsc.VectorSubcoreMesh(core_axis_name="c", subcore_axis_name="s")

    @functools.partial(
        pl.kernel,
        mesh=mesh,
        out_type=jax.ShapeDtypeStruct((_BATCH,), jnp.float32),
        scratch_types=[
            pltpu.VMEM((_B_PER_W,), jnp.int32),
            pltpu.VMEM((_B_PER_W,), jnp.int32),
            pltpu.VMEM((_B_PER_W, _D), jnp.float32),
            pltpu.VMEM((_B_PER_W, _D), jnp.float32),
            pltpu.VMEM((_B_PER_W,), jnp.float32),
            pltpu.SemaphoreType.DMA,
            pltpu.SemaphoreType.DMA,
        ],
        compiler_params=_compiler_params(),
    )
    def sc_kernel(uid_hbm, iid_hbm, ut_hbm, it_hbm, out_hbm,
                  uidx_v, iidx_v, u_rows, i_rows, out_v, sem_u, sem_i):
        wid = lax.axis_index("s") * _NC + lax.axis_index("c")
        base = wid * _B_PER_W
        pltpu.sync_copy(uid_hbm.at[pl.ds(base, _B_PER_W)], uidx_v)
        pltpu.sync_copy(iid_hbm.at[pl.ds(base, _B_PER_W)], iidx_v)
        cu = pltpu.async_copy(ut_hbm.at[uidx_v], u_rows, sem_u)
        ci = pltpu.async_copy(it_hbm.at[iidx_v], i_rows, sem_i)
        cu.wait()
        ci.wait()

        lane = lax.iota(jnp.int32, _L)

        @pl.loop(0, _B_PER_W, step=_L)
        def _(g):
            rows = g + lane
            acc = jnp.zeros((_L,), jnp.float32)
            for d in range(_D):
                col = jnp.full((_L,), d, jnp.int32)
                ug = plsc.load_gather(u_rows, [rows, col])
                vg = plsc.load_gather(i_rows, [rows, col])
                acc = acc + ug * vg
            out_v[pl.ds(g, _L)] = acc

        pltpu.sync_copy(out_v, out_hbm.at[pl.ds(base, _B_PER_W)])

    return sc_kernel(user_ids, item_ids, user_table, item_table)
